# baseline (speedup 1.0000x reference)
import functools

import numpy as np
import jax
import jax.numpy as jnp
from jax import lax
from jax.experimental import pallas as pl
from jax.experimental.pallas import tpu as pltpu
from jax.experimental.pallas import tpu_sc as plsc

B, S, H, K, P = 16, 128, 300, 12, 50
D = 2 * K + 1


def _build_static():
    base = np.arange(1, S + 1)
    emo = np.repeat(base, S)
    cau = np.tile(base, S)
    rel = cau - emo
    m = np.abs(rel) <= K
    emo_i = emo[m] - 1
    cau_i = cau[m] - 1
    rel_i = rel[m]
    n_pair = int(emo_i.shape[0])
    cnt = np.bincount(rel_i + K, minlength=D).astype(np.float32)
    u = np.arange(D, dtype=np.float32)
    g = cnt[None, :] * np.exp(-((u[:, None] - u[None, :]) ** 2))
    sel = ((rel_i + K) * S + emo_i).astype(np.int64)
    pos_pairs = np.stack([emo_i + 1, cau_i + 1], axis=1).astype(np.int32)
    return n_pair, g, sel, pos_pairs


N_PAIR, G_MAT, _SEL, POS_PAIRS = _build_static()
N_PAD = 3072
_NW = 32
_CHUNK = (B * N_PAD) // _NW

BAND_STRIDE = 4096

_sel_pad = np.zeros((N_PAD,), dtype=np.int64)
_sel_pad[:N_PAIR] = _SEL
IDX_FLAT = (np.arange(B, dtype=np.int64)[:, None] * BAND_STRIDE
            + _sel_pad[None, :]).reshape(-1).astype(np.int32)


BB = 4


def _band_body(gm_ref, he_ref, hc_ref, hs_ref, pos_ref, w_ref, bh_ref, g_ref,
               b_ref, wr_ref, br_ref, out_ref):
    w = w_ref[...]
    e_tab = jnp.dot(
        jnp.dot(gm_ref[...], pos_ref[...], preferred_element_type=jnp.float32),
        w[2 * H:2 * H + P, :], preferred_element_type=jnp.float32,
    ) + bh_ref[...]
    g_row = g_ref[...]
    b_row = b_ref[...]
    wr_row = wr_ref[...].reshape(1, H)
    br = br_ref[0, 0]
    for bb in range(BB):
        hs = hs_ref[bb]
        x1 = he_ref[bb] + hs
        x2 = hc_ref[bb] + hs
        a = jnp.dot(x1, w[0:H, :], preferred_element_type=jnp.float32)
        c = jnp.dot(x2, w[H:2 * H, :], preferred_element_type=jnp.float32)
        for d in range(D):
            o = (d - K) % S
            cs = jnp.concatenate([c[o:], c[:o]], axis=0) if o else c
            v = a + cs + e_tab[d, :][None, :]
            mu = jnp.mean(v, axis=-1, keepdims=True)
            vc = v - mu
            var = jnp.mean(vc * vc, axis=-1, keepdims=True)
            y = vc * lax.rsqrt(var + 1e-5) * g_row + b_row
            y = jnp.where(y > 0, y, jnp.exp(jnp.minimum(y, 0.0)) - 1.0)
            out_ref[bb, d, :] = jnp.sum(y * wr_row, axis=-1) + br


_band_call = pl.pallas_call(
    _band_body,
    grid=(B // BB,),
    in_specs=[
        pl.BlockSpec((D, D), lambda b: (0, 0)),
        pl.BlockSpec((BB, S, H), lambda b: (b, 0, 0)),
        pl.BlockSpec((BB, S, H), lambda b: (b, 0, 0)),
        pl.BlockSpec((BB, S, H), lambda b: (b, 0, 0)),
        pl.BlockSpec((D, P), lambda b: (0, 0)),
        pl.BlockSpec((2 * H + P, H), lambda b: (0, 0)),
        pl.BlockSpec((1, H), lambda b: (0, 0)),
        pl.BlockSpec((1, H), lambda b: (0, 0)),
        pl.BlockSpec((1, H), lambda b: (0, 0)),
        pl.BlockSpec((H, 1), lambda b: (0, 0)),
        pl.BlockSpec((1, 1), lambda b: (0, 0)),
    ],
    out_specs=pl.BlockSpec((BB, BAND_STRIDE // S, S), lambda b: (b, 0, 0)),
    out_shape=jax.ShapeDtypeStruct((B, BAND_STRIDE // S, S), jnp.float32),
    compiler_params=pltpu.CompilerParams(
        dimension_semantics=("arbitrary",),
    ),
)


def _gather_body(tab_hbm, idx_hbm, out_hbm, idx_v, rows_v, sem):
    wid = lax.axis_index("s") * 2 + lax.axis_index("c")
    base = wid * _CHUNK
    pltpu.sync_copy(idx_hbm.at[pl.ds(base, _CHUNK)], idx_v)
    pltpu.async_copy(tab_hbm.at[idx_v], rows_v, sem).wait()
    pltpu.sync_copy(rows_v, out_hbm.at[pl.ds(base, _CHUNK)])


@functools.cache
def _gather_call():
    return pl.kernel(
        _gather_body,
        mesh=plsc.VectorSubcoreMesh(core_axis_name="c", subcore_axis_name="s"),
        out_type=jax.ShapeDtypeStruct((B * N_PAD,), jnp.float32),
        scratch_types=[
            pltpu.VMEM((_CHUNK,), jnp.int32),
            pltpu.VMEM((_CHUNK,), jnp.float32),
            pltpu.SemaphoreType.DMA,
        ],
    )


def kernel(h_e, h_c, h_share, pos_emb, W_hid, b_hid, ln_g, ln_b, W_rel, b_rel):
    band = _band_call(
        jnp.asarray(G_MAT, dtype=jnp.float32),
        h_e, h_c, h_share, pos_emb, W_hid,
        b_hid.reshape(1, H), ln_g.reshape(1, H), ln_b.reshape(1, H),
        W_rel, b_rel.reshape(1, 1),
    )
    flat = _gather_call()(band.reshape(B * BAND_STRIDE), jnp.asarray(IDX_FLAT))
    out = flat.reshape(B, N_PAD)[:, :N_PAIR]
    return out, jnp.asarray(POS_PAIRS)


# speedup vs baseline: 1.1587x; 1.1587x over previous
import functools

import numpy as np
import jax
import jax.numpy as jnp
from jax import lax
from jax.experimental import pallas as pl
from jax.experimental.pallas import tpu as pltpu
from jax.experimental.pallas import tpu_sc as plsc

B, S, H, K, P = 16, 128, 300, 12, 50
D = 2 * K + 1


def _build_static():
    base = np.arange(1, S + 1)
    emo = np.repeat(base, S)
    cau = np.tile(base, S)
    rel = cau - emo
    m = np.abs(rel) <= K
    emo_i = emo[m] - 1
    cau_i = cau[m] - 1
    rel_i = rel[m]
    n_pair = int(emo_i.shape[0])
    cnt = np.bincount(rel_i + K, minlength=D).astype(np.float32)
    u = np.arange(D, dtype=np.float32)
    g = cnt[None, :] * np.exp(-((u[:, None] - u[None, :]) ** 2))
    sel = ((rel_i + K) * S + emo_i).astype(np.int64)
    pos_pairs = np.stack([emo_i + 1, cau_i + 1], axis=1).astype(np.int32)
    return n_pair, g, sel, pos_pairs


N_PAIR, G_MAT, _SEL, POS_PAIRS = _build_static()
N_PAD = 3072
_NW = 32
_CHUNK = (B * N_PAD) // _NW

BAND_STRIDE = 4096

_sel_pad = np.zeros((N_PAD,), dtype=np.int64)
_sel_pad[:N_PAIR] = _SEL
IDX_FLAT = (np.arange(B, dtype=np.int64)[:, None] * BAND_STRIDE
            + _sel_pad[None, :]).reshape(-1).astype(np.int32)


BB = 1


def _band_body(gm_ref, he_ref, hc_ref, hs_ref, pos_ref, w_ref, bh_ref, g_ref,
               b_ref, wr_ref, br_ref, out_ref):
    w = w_ref[...]
    e_tab = jnp.dot(
        jnp.dot(gm_ref[...], pos_ref[...], preferred_element_type=jnp.float32),
        w[2 * H:2 * H + P, :], preferred_element_type=jnp.float32,
    ) + bh_ref[...]
    g_row = g_ref[...]
    b_row = b_ref[...]
    wr_row = wr_ref[...].reshape(1, H)
    br = br_ref[0, 0]
    g16 = g_row.astype(jnp.bfloat16)
    b16 = b_row.astype(jnp.bfloat16)
    w16 = wr_row.astype(jnp.bfloat16)
    one16 = jnp.bfloat16(1.0)
    for bb in range(BB):
        hs = hs_ref[bb]
        x1 = he_ref[bb] + hs
        x2 = hc_ref[bb] + hs
        a = jnp.dot(x1, w[0:H, :], preferred_element_type=jnp.float32)
        c = jnp.dot(x2, w[H:2 * H, :], preferred_element_type=jnp.float32)
        for d in range(D):
            o = (d - K) % S
            cs = jnp.concatenate([c[o:], c[:o]], axis=0) if o else c
            v = a + cs + e_tab[d, :][None, :]
            mu = jnp.mean(v, axis=-1, keepdims=True)
            vc = v - mu
            var = jnp.mean(vc * vc, axis=-1, keepdims=True)
            t16 = (vc * lax.rsqrt(var + 1e-5)).astype(jnp.bfloat16)
            y = t16 * g16 + b16
            y = jnp.where(y > 0, y, jnp.exp(y) - one16)
            out_ref[bb, d, :] = jnp.sum((y * w16).astype(jnp.float32),
                                        axis=-1) + br


_band_call = pl.pallas_call(
    _band_body,
    grid=(B // BB,),
    in_specs=[
        pl.BlockSpec((D, D), lambda b: (0, 0)),
        pl.BlockSpec((BB, S, H), lambda b: (b, 0, 0)),
        pl.BlockSpec((BB, S, H), lambda b: (b, 0, 0)),
        pl.BlockSpec((BB, S, H), lambda b: (b, 0, 0)),
        pl.BlockSpec((D, P), lambda b: (0, 0)),
        pl.BlockSpec((2 * H + P, H), lambda b: (0, 0)),
        pl.BlockSpec((1, H), lambda b: (0, 0)),
        pl.BlockSpec((1, H), lambda b: (0, 0)),
        pl.BlockSpec((1, H), lambda b: (0, 0)),
        pl.BlockSpec((H, 1), lambda b: (0, 0)),
        pl.BlockSpec((1, 1), lambda b: (0, 0)),
    ],
    out_specs=pl.BlockSpec((BB, BAND_STRIDE // S, S), lambda b: (b, 0, 0)),
    out_shape=jax.ShapeDtypeStruct((B, BAND_STRIDE // S, S), jnp.float32),
    compiler_params=pltpu.CompilerParams(
        dimension_semantics=("arbitrary",),
    ),
)


def _gather_body(tab_hbm, idx_hbm, out_hbm, idx_v, rows_v, sem):
    wid = lax.axis_index("s") * 2 + lax.axis_index("c")
    base = wid * _CHUNK
    pltpu.sync_copy(idx_hbm.at[pl.ds(base, _CHUNK)], idx_v)
    pltpu.async_copy(tab_hbm.at[idx_v], rows_v, sem).wait()
    pltpu.sync_copy(rows_v, out_hbm.at[pl.ds(base, _CHUNK)])


@functools.cache
def _gather_call():
    return pl.kernel(
        _gather_body,
        mesh=plsc.VectorSubcoreMesh(core_axis_name="c", subcore_axis_name="s"),
        out_type=jax.ShapeDtypeStruct((B * N_PAD,), jnp.float32),
        scratch_types=[
            pltpu.VMEM((_CHUNK,), jnp.int32),
            pltpu.VMEM((_CHUNK,), jnp.float32),
            pltpu.SemaphoreType.DMA,
        ],
    )


def kernel(h_e, h_c, h_share, pos_emb, W_hid, b_hid, ln_g, ln_b, W_rel, b_rel):
    band = _band_call(
        jnp.asarray(G_MAT, dtype=jnp.float32),
        h_e, h_c, h_share, pos_emb, W_hid,
        b_hid.reshape(1, H), ln_g.reshape(1, H), ln_b.reshape(1, H),
        W_rel, b_rel.reshape(1, 1),
    )
    flat = _gather_call()(band.reshape(B * BAND_STRIDE), jnp.asarray(IDX_FLAT))
    out = flat.reshape(B, N_PAD)[:, :N_PAIR]
    return out, jnp.asarray(POS_PAIRS)


# speedup vs baseline: 1.4067x; 1.2140x over previous
import functools

import numpy as np
import jax
import jax.numpy as jnp
from jax import lax
from jax.experimental import pallas as pl
from jax.experimental.pallas import tpu as pltpu
from jax.experimental.pallas import tpu_sc as plsc

B, S, H, K, P = 16, 128, 300, 12, 50
D = 2 * K + 1


def _build_static():
    base = np.arange(1, S + 1)
    emo = np.repeat(base, S)
    cau = np.tile(base, S)
    rel = cau - emo
    m = np.abs(rel) <= K
    emo_i = emo[m] - 1
    cau_i = cau[m] - 1
    rel_i = rel[m]
    n_pair = int(emo_i.shape[0])
    cnt = np.bincount(rel_i + K, minlength=D).astype(np.float32)
    u = np.arange(D, dtype=np.float32)
    g = cnt[None, :] * np.exp(-((u[:, None] - u[None, :]) ** 2))
    sel = ((rel_i + K) * S + emo_i).astype(np.int64)
    pos_pairs = np.stack([emo_i + 1, cau_i + 1], axis=1).astype(np.int32)
    return n_pair, g, sel, pos_pairs


N_PAIR, G_MAT, _SEL, POS_PAIRS = _build_static()
N_PAD = 3072
_NW = 32
_CHUNK = (B * N_PAD) // _NW

BAND_STRIDE = 4096

_sel_pad = np.zeros((N_PAD,), dtype=np.int64)
_sel_pad[:N_PAIR] = _SEL
IDX_FLAT = (np.arange(B, dtype=np.int64)[:, None] * BAND_STRIDE
            + _sel_pad[None, :]).reshape(-1).astype(np.int32)


BB = 1


def _band_body(gm_ref, he_ref, hc_ref, hs_ref, pos_ref, w_ref, bh_ref, g_ref,
               b_ref, wr_ref, br_ref, out_ref):
    w = w_ref[...]
    e_tab = jnp.dot(
        jnp.dot(gm_ref[...], pos_ref[...], preferred_element_type=jnp.float32),
        w[2 * H:2 * H + P, :], preferred_element_type=jnp.float32,
    ) + bh_ref[...]
    g_row = g_ref[...]
    b_row = b_ref[...]
    wr_row = wr_ref[...].reshape(1, H)
    br = br_ref[0, 0]
    g16 = g_row.astype(jnp.bfloat16)
    b16 = b_row.astype(jnp.bfloat16)
    w16 = wr_row.astype(jnp.bfloat16)
    one16 = jnp.bfloat16(1.0)
    ones = jnp.ones((H, 1), dtype=jnp.float32)
    inv_h = 1.0 / H
    se = jnp.dot(e_tab, ones, preferred_element_type=jnp.float32)
    w16c = wr_ref[...].astype(jnp.bfloat16)
    for bb in range(BB):
        hs = hs_ref[bb]
        x1 = he_ref[bb] + hs
        x2 = hc_ref[bb] + hs
        a = jnp.dot(x1, w[0:H, :], preferred_element_type=jnp.float32)
        c = jnp.dot(x2, w[H:2 * H, :], preferred_element_type=jnp.float32)
        t_cols = []
        for d in range(D):
            o = (d - K) % S
            cs = jnp.concatenate([c[o:], c[:o]], axis=0) if o else c
            v = a + cs + e_tab[d, :][None, :]
            mu = jnp.mean(v, axis=-1, keepdims=True)
            vc = v - mu
            var = jnp.mean(vc * vc, axis=-1, keepdims=True)
            t16 = (vc * lax.rsqrt(var + 1e-5)).astype(jnp.bfloat16)
            y = t16 * g16 + b16
            y = jnp.where(y > 0, y, jnp.exp(y) - one16)
            t_cols.append(jnp.dot(y, w16c, preferred_element_type=jnp.float32))
        t_all = jnp.transpose(jnp.concatenate(t_cols, axis=1))
        out_ref[bb, 0:D, :] = t_all + br


_band_call = pl.pallas_call(
    _band_body,
    grid=(B // BB,),
    in_specs=[
        pl.BlockSpec((D, D), lambda b: (0, 0)),
        pl.BlockSpec((BB, S, H), lambda b: (b, 0, 0)),
        pl.BlockSpec((BB, S, H), lambda b: (b, 0, 0)),
        pl.BlockSpec((BB, S, H), lambda b: (b, 0, 0)),
        pl.BlockSpec((D, P), lambda b: (0, 0)),
        pl.BlockSpec((2 * H + P, H), lambda b: (0, 0)),
        pl.BlockSpec((1, H), lambda b: (0, 0)),
        pl.BlockSpec((1, H), lambda b: (0, 0)),
        pl.BlockSpec((1, H), lambda b: (0, 0)),
        pl.BlockSpec((H, 1), lambda b: (0, 0)),
        pl.BlockSpec((1, 1), lambda b: (0, 0)),
    ],
    out_specs=pl.BlockSpec((BB, BAND_STRIDE // S, S), lambda b: (b, 0, 0)),
    out_shape=jax.ShapeDtypeStruct((B, BAND_STRIDE // S, S), jnp.float32),
    compiler_params=pltpu.CompilerParams(
        dimension_semantics=("arbitrary",),
    ),
)


def _gather_body(tab_hbm, idx_hbm, out_hbm, idx_v, rows_v, sem):
    wid = lax.axis_index("s") * 2 + lax.axis_index("c")
    base = wid * _CHUNK
    pltpu.sync_copy(idx_hbm.at[pl.ds(base, _CHUNK)], idx_v)
    pltpu.async_copy(tab_hbm.at[idx_v], rows_v, sem).wait()
    pltpu.sync_copy(rows_v, out_hbm.at[pl.ds(base, _CHUNK)])


@functools.cache
def _gather_call():
    return pl.kernel(
        _gather_body,
        mesh=plsc.VectorSubcoreMesh(core_axis_name="c", subcore_axis_name="s"),
        out_type=jax.ShapeDtypeStruct((B * N_PAD,), jnp.float32),
        scratch_types=[
            pltpu.VMEM((_CHUNK,), jnp.int32),
            pltpu.VMEM((_CHUNK,), jnp.float32),
            pltpu.SemaphoreType.DMA,
        ],
    )


def kernel(h_e, h_c, h_share, pos_emb, W_hid, b_hid, ln_g, ln_b, W_rel, b_rel):
    band = _band_call(
        jnp.asarray(G_MAT, dtype=jnp.float32),
        h_e, h_c, h_share, pos_emb, W_hid,
        b_hid.reshape(1, H), ln_g.reshape(1, H), ln_b.reshape(1, H),
        W_rel, b_rel.reshape(1, 1),
    )
    flat = _gather_call()(band.reshape(B * BAND_STRIDE), jnp.asarray(IDX_FLAT))
    out = flat.reshape(B, N_PAD)[:, :N_PAIR]
    return out, jnp.asarray(POS_PAIRS)


# speedup vs baseline: 1.4111x; 1.0031x over previous
import functools

import numpy as np
import jax
import jax.numpy as jnp
from jax import lax
from jax.experimental import pallas as pl
from jax.experimental.pallas import tpu as pltpu
from jax.experimental.pallas import tpu_sc as plsc

B, S, H, K, P = 16, 128, 300, 12, 50
D = 2 * K + 1


def _build_static():
    base = np.arange(1, S + 1)
    emo = np.repeat(base, S)
    cau = np.tile(base, S)
    rel = cau - emo
    m = np.abs(rel) <= K
    emo_i = emo[m] - 1
    cau_i = cau[m] - 1
    rel_i = rel[m]
    n_pair = int(emo_i.shape[0])
    cnt = np.bincount(rel_i + K, minlength=D).astype(np.float32)
    u = np.arange(D, dtype=np.float32)
    g = cnt[None, :] * np.exp(-((u[:, None] - u[None, :]) ** 2))
    sel = ((rel_i + K) * S + emo_i).astype(np.int64)
    pos_pairs = np.stack([emo_i + 1, cau_i + 1], axis=1).astype(np.int32)
    return n_pair, g, sel, pos_pairs


N_PAIR, G_MAT, _SEL, POS_PAIRS = _build_static()
N_PAD = 3072
_NW = 32
_CHUNK = (B * N_PAD) // _NW

BAND_STRIDE = 4096

_sel_pad = np.zeros((N_PAD,), dtype=np.int64)
_sel_pad[:N_PAIR] = _SEL
IDX_FLAT = (np.arange(B, dtype=np.int64)[:, None] * BAND_STRIDE
            + _sel_pad[None, :]).reshape(-1).astype(np.int32)


BB = 1


def _band_body(gm_ref, he_ref, hc_ref, pos_ref, w_ref, bh_ref, g_ref,
               b_ref, wr_ref, br_ref, out_ref):
    w = w_ref[...]
    e_tab = jnp.dot(
        jnp.dot(gm_ref[...], pos_ref[...], preferred_element_type=jnp.float32),
        w[2 * H:2 * H + P, :], preferred_element_type=jnp.float32,
    ) + bh_ref[...]
    g_row = g_ref[...]
    b_row = b_ref[...]
    wr_row = wr_ref[...].reshape(1, H)
    br = br_ref[0, 0]
    g16 = g_row.astype(jnp.bfloat16)
    b16 = b_row.astype(jnp.bfloat16)
    w16 = wr_row.astype(jnp.bfloat16)
    one16 = jnp.bfloat16(1.0)
    ones = jnp.ones((H, 1), dtype=jnp.float32)
    inv_h = 1.0 / H
    se = jnp.dot(e_tab, ones, preferred_element_type=jnp.float32)
    w16c = wr_ref[...].astype(jnp.bfloat16)
    for bb in range(BB):
        x1 = he_ref[bb]
        x2 = hc_ref[bb]
        a = jnp.dot(x1, w[0:H, :], preferred_element_type=jnp.float32)
        c = jnp.dot(x2, w[H:2 * H, :], preferred_element_type=jnp.float32)
        t_cols = []
        for d in range(D):
            o = (d - K) % S
            cs = jnp.concatenate([c[o:], c[:o]], axis=0) if o else c
            v = a + cs + e_tab[d, :][None, :]
            mu = jnp.mean(v, axis=-1, keepdims=True)
            vc = v - mu
            var = jnp.mean(vc * vc, axis=-1, keepdims=True)
            t16 = (vc * lax.rsqrt(var + 1e-5)).astype(jnp.bfloat16)
            y = t16 * g16 + b16
            y = jnp.where(y > 0, y, jnp.exp(y) - one16)
            t_cols.append(jnp.dot(y, w16c, preferred_element_type=jnp.float32))
        t_all = jnp.transpose(jnp.concatenate(t_cols, axis=1))
        out_ref[bb, 0:D, :] = t_all + br


_band_call = pl.pallas_call(
    _band_body,
    grid=(B // BB,),
    in_specs=[
        pl.BlockSpec((D, D), lambda b: (0, 0)),
        pl.BlockSpec((BB, S, H), lambda b: (b, 0, 0)),
        pl.BlockSpec((BB, S, H), lambda b: (b, 0, 0)),
        pl.BlockSpec((D, P), lambda b: (0, 0)),
        pl.BlockSpec((2 * H + P, H), lambda b: (0, 0)),
        pl.BlockSpec((1, H), lambda b: (0, 0)),
        pl.BlockSpec((1, H), lambda b: (0, 0)),
        pl.BlockSpec((1, H), lambda b: (0, 0)),
        pl.BlockSpec((H, 1), lambda b: (0, 0)),
        pl.BlockSpec((1, 1), lambda b: (0, 0)),
    ],
    out_specs=pl.BlockSpec((BB, BAND_STRIDE // S, S), lambda b: (b, 0, 0)),
    out_shape=jax.ShapeDtypeStruct((B, BAND_STRIDE // S, S), jnp.float32),
    compiler_params=pltpu.CompilerParams(
        dimension_semantics=("arbitrary",),
    ),
)


def _gather_body(tab_hbm, idx_hbm, out_hbm, idx_v, rows_v, sem):
    wid = lax.axis_index("s") * 2 + lax.axis_index("c")
    base = wid * _CHUNK
    pltpu.sync_copy(idx_hbm.at[pl.ds(base, _CHUNK)], idx_v)
    pltpu.async_copy(tab_hbm.at[idx_v], rows_v, sem).wait()
    pltpu.sync_copy(rows_v, out_hbm.at[pl.ds(base, _CHUNK)])


@functools.cache
def _gather_call():
    return pl.kernel(
        _gather_body,
        mesh=plsc.VectorSubcoreMesh(core_axis_name="c", subcore_axis_name="s"),
        out_type=jax.ShapeDtypeStruct((B * N_PAD,), jnp.float32),
        scratch_types=[
            pltpu.VMEM((_CHUNK,), jnp.int32),
            pltpu.VMEM((_CHUNK,), jnp.float32),
            pltpu.SemaphoreType.DMA,
        ],
    )


def kernel(h_e, h_c, h_share, pos_emb, W_hid, b_hid, ln_g, ln_b, W_rel, b_rel):
    band = _band_call(
        jnp.asarray(G_MAT, dtype=jnp.float32),
        h_e + h_share, h_c + h_share, pos_emb, W_hid,
        b_hid.reshape(1, H), ln_g.reshape(1, H), ln_b.reshape(1, H),
        W_rel, b_rel.reshape(1, 1),
    )
    flat = _gather_call()(band.reshape(B * BAND_STRIDE), jnp.asarray(IDX_FLAT))
    out = flat.reshape(B, N_PAD)[:, :N_PAIR]
    return out, jnp.asarray(POS_PAIRS)


# speedup vs baseline: 1.4457x; 1.0245x over previous
import functools

import numpy as np
import jax
import jax.numpy as jnp
from jax import lax
from jax.experimental import pallas as pl
from jax.experimental.pallas import tpu as pltpu
from jax.experimental.pallas import tpu_sc as plsc

B, S, H, K, P = 16, 128, 300, 12, 50
D = 2 * K + 1


def _build_static():
    base = np.arange(1, S + 1)
    emo = np.repeat(base, S)
    cau = np.tile(base, S)
    rel = cau - emo
    m = np.abs(rel) <= K
    emo_i = emo[m] - 1
    cau_i = cau[m] - 1
    rel_i = rel[m]
    n_pair = int(emo_i.shape[0])
    cnt = np.bincount(rel_i + K, minlength=D).astype(np.float32)
    u = np.arange(D, dtype=np.float32)
    g = cnt[None, :] * np.exp(-((u[:, None] - u[None, :]) ** 2))
    sel = ((rel_i + K) * S + emo_i).astype(np.int64)
    pos_pairs = np.stack([emo_i + 1, cau_i + 1], axis=1).astype(np.int32)
    return n_pair, g, sel, pos_pairs


N_PAIR, G_MAT, _SEL, POS_PAIRS = _build_static()
N_PAD = 3072
_NW = 32
_CHUNK = (B * N_PAD) // _NW

BAND_STRIDE = 4096

_sel_pad = np.zeros((N_PAD,), dtype=np.int64)
_sel_pad[:N_PAIR] = _SEL
IDX_FLAT = (np.arange(B, dtype=np.int64)[:, None] * BAND_STRIDE
            + _sel_pad[None, :]).reshape(-1).astype(np.int32)


BB = 2


def _band_body(gm_ref, he_ref, hc_ref, pos_ref, w_ref, bh_ref, g_ref,
               b_ref, wr_ref, br_ref, out_ref):
    w = w_ref[...]
    e_tab = jnp.dot(
        jnp.dot(gm_ref[...], pos_ref[...], preferred_element_type=jnp.float32),
        w[2 * H:2 * H + P, :], preferred_element_type=jnp.float32,
    ) + bh_ref[...]
    g_row = g_ref[...]
    b_row = b_ref[...]
    wr_row = wr_ref[...].reshape(1, H)
    br = br_ref[0, 0]
    g16 = g_row.astype(jnp.bfloat16)
    b16 = b_row.astype(jnp.bfloat16)
    w16 = wr_row.astype(jnp.bfloat16)
    one16 = jnp.bfloat16(1.0)
    ones = jnp.ones((H, 1), dtype=jnp.float32)
    inv_h = 1.0 / H
    se = jnp.dot(e_tab, ones, preferred_element_type=jnp.float32)
    w16c = wr_ref[...].astype(jnp.bfloat16)
    for bb in range(BB):
        x1 = he_ref[bb]
        x2 = hc_ref[bb]
        a = jnp.dot(x1, w[0:H, :], preferred_element_type=jnp.float32)
        c = jnp.dot(x2, w[H:2 * H, :], preferred_element_type=jnp.float32)
        t_cols = []
        for d in range(D):
            o = (d - K) % S
            cs = jnp.concatenate([c[o:], c[:o]], axis=0) if o else c
            v = a + cs + e_tab[d, :][None, :]
            mu = jnp.mean(v, axis=-1, keepdims=True)
            vc = v - mu
            var = jnp.mean(vc * vc, axis=-1, keepdims=True)
            t16 = (vc * lax.rsqrt(var + 1e-5)).astype(jnp.bfloat16)
            y = t16 * g16 + b16
            y = jnp.where(y > 0, y, jnp.exp(y) - one16)
            t_cols.append(jnp.dot(y, w16c, preferred_element_type=jnp.float32))
        t_all = jnp.transpose(jnp.concatenate(t_cols, axis=1))
        out_ref[bb, 0:D, :] = t_all + br


_band_call = pl.pallas_call(
    _band_body,
    grid=(B // BB,),
    in_specs=[
        pl.BlockSpec((D, D), lambda b: (0, 0)),
        pl.BlockSpec((BB, S, H), lambda b: (b, 0, 0)),
        pl.BlockSpec((BB, S, H), lambda b: (b, 0, 0)),
        pl.BlockSpec((D, P), lambda b: (0, 0)),
        pl.BlockSpec((2 * H + P, H), lambda b: (0, 0)),
        pl.BlockSpec((1, H), lambda b: (0, 0)),
        pl.BlockSpec((1, H), lambda b: (0, 0)),
        pl.BlockSpec((1, H), lambda b: (0, 0)),
        pl.BlockSpec((H, 1), lambda b: (0, 0)),
        pl.BlockSpec((1, 1), lambda b: (0, 0)),
    ],
    out_specs=pl.BlockSpec((BB, BAND_STRIDE // S, S), lambda b: (b, 0, 0)),
    out_shape=jax.ShapeDtypeStruct((B, BAND_STRIDE // S, S), jnp.float32),
    compiler_params=pltpu.CompilerParams(
        dimension_semantics=("arbitrary",),
    ),
)


def _gather_body(tab_hbm, idx_hbm, out_hbm, idx_v, rows_v, sem):
    wid = lax.axis_index("s") * 2 + lax.axis_index("c")
    base = wid * _CHUNK
    pltpu.sync_copy(idx_hbm.at[pl.ds(base, _CHUNK)], idx_v)
    pltpu.async_copy(tab_hbm.at[idx_v], rows_v, sem).wait()
    pltpu.sync_copy(rows_v, out_hbm.at[pl.ds(base, _CHUNK)])


@functools.cache
def _gather_call():
    return pl.kernel(
        _gather_body,
        mesh=plsc.VectorSubcoreMesh(core_axis_name="c", subcore_axis_name="s"),
        out_type=jax.ShapeDtypeStruct((B * N_PAD,), jnp.float32),
        scratch_types=[
            pltpu.VMEM((_CHUNK,), jnp.int32),
            pltpu.VMEM((_CHUNK,), jnp.float32),
            pltpu.SemaphoreType.DMA,
        ],
    )


def kernel(h_e, h_c, h_share, pos_emb, W_hid, b_hid, ln_g, ln_b, W_rel, b_rel):
    band = _band_call(
        jnp.asarray(G_MAT, dtype=jnp.float32),
        h_e + h_share, h_c + h_share, pos_emb, W_hid,
        b_hid.reshape(1, H), ln_g.reshape(1, H), ln_b.reshape(1, H),
        W_rel, b_rel.reshape(1, 1),
    )
    flat = _gather_call()(band.reshape(B * BAND_STRIDE), jnp.asarray(IDX_FLAT))
    out = flat.reshape(B, N_PAD)[:, :N_PAIR]
    return out, jnp.asarray(POS_PAIRS)


# speedup vs baseline: 1.5945x; 1.1029x over previous
import functools

import numpy as np
import jax
import jax.numpy as jnp
from jax import lax
from jax.experimental import pallas as pl
from jax.experimental.pallas import tpu as pltpu
from jax.experimental.pallas import tpu_sc as plsc

B, S, H, K, P = 16, 128, 300, 12, 50
D = 2 * K + 1


def _build_static():
    base = np.arange(1, S + 1)
    emo = np.repeat(base, S)
    cau = np.tile(base, S)
    rel = cau - emo
    m = np.abs(rel) <= K
    emo_i = emo[m] - 1
    cau_i = cau[m] - 1
    rel_i = rel[m]
    n_pair = int(emo_i.shape[0])
    cnt = np.bincount(rel_i + K, minlength=D).astype(np.float32)
    u = np.arange(D, dtype=np.float32)
    g = cnt[None, :] * np.exp(-((u[:, None] - u[None, :]) ** 2))
    sel = ((rel_i + K) * S + emo_i).astype(np.int64)
    pos_pairs = np.stack([emo_i + 1, cau_i + 1], axis=1).astype(np.int32)
    return n_pair, g, sel, pos_pairs


N_PAIR, G_MAT, _SEL, POS_PAIRS = _build_static()
N_PAD = 3072
_NW = 32
_CHUNK = (B * N_PAD) // _NW

BAND_STRIDE = 4096

_sel_pad = np.zeros((N_PAD,), dtype=np.int64)
_sel_pad[:N_PAIR] = _SEL
IDX_FLAT = (np.arange(B, dtype=np.int64)[:, None] * BAND_STRIDE
            + _sel_pad[None, :]).reshape(-1).astype(np.int32)


BB = 2


def _band_body(gm_ref, he_ref, hc_ref, pos_ref, w_ref, bh_ref, g_ref,
               b_ref, wr_ref, br_ref, out_ref):
    w = w_ref[...]
    e_tab = jnp.dot(
        jnp.dot(gm_ref[...], pos_ref[...], preferred_element_type=jnp.float32),
        w[2 * H:2 * H + P, :], preferred_element_type=jnp.float32,
    ) + bh_ref[...]
    g_row = g_ref[...]
    b_row = b_ref[...]
    wr_row = wr_ref[...].reshape(1, H)
    br = br_ref[0, 0]
    g16 = g_row.astype(jnp.bfloat16)
    b16 = b_row.astype(jnp.bfloat16)
    w16 = wr_row.astype(jnp.bfloat16)
    one16 = jnp.bfloat16(1.0)
    ones = jnp.ones((H, 1), dtype=jnp.float32)
    inv_h = 1.0 / H
    se = jnp.dot(e_tab, ones, preferred_element_type=jnp.float32)
    w16c = wr_ref[...].astype(jnp.bfloat16)
    for bb in range(BB):
        x1 = he_ref[bb]
        x2 = hc_ref[bb]
        a = jnp.dot(x1, w[0:H, :], preferred_element_type=jnp.float32)
        c = jnp.dot(x2, w[H:2 * H, :], preferred_element_type=jnp.float32)
        for d in range(D):
            o = (d - K) % S
            cs = jnp.concatenate([c[o:], c[:o]], axis=0) if o else c
            v = a + cs + e_tab[d, :][None, :]
            m1 = jnp.sum(v, axis=-1, keepdims=True)
            m2 = jnp.sum(v * v, axis=-1, keepdims=True)
            mu = m1 * inv_h
            var = m2 * inv_h - mu * mu
            t16 = ((v - mu) * lax.rsqrt(var + 1e-5)).astype(jnp.bfloat16)
            y = t16 * g16 + b16
            y = jnp.where(y > 0, y, jnp.exp(y) - one16)
            t_row = lax.dot_general(w16, y, (((1,), (1,)), ((), ())),
                                    preferred_element_type=jnp.float32)
            out_ref[bb, d, :] = t_row[0] + br


_band_call = pl.pallas_call(
    _band_body,
    grid=(B // BB,),
    in_specs=[
        pl.BlockSpec((D, D), lambda b: (0, 0)),
        pl.BlockSpec((BB, S, H), lambda b: (b, 0, 0)),
        pl.BlockSpec((BB, S, H), lambda b: (b, 0, 0)),
        pl.BlockSpec((D, P), lambda b: (0, 0)),
        pl.BlockSpec((2 * H + P, H), lambda b: (0, 0)),
        pl.BlockSpec((1, H), lambda b: (0, 0)),
        pl.BlockSpec((1, H), lambda b: (0, 0)),
        pl.BlockSpec((1, H), lambda b: (0, 0)),
        pl.BlockSpec((H, 1), lambda b: (0, 0)),
        pl.BlockSpec((1, 1), lambda b: (0, 0)),
    ],
    out_specs=pl.BlockSpec((BB, BAND_STRIDE // S, S), lambda b: (b, 0, 0)),
    out_shape=jax.ShapeDtypeStruct((B, BAND_STRIDE // S, S), jnp.float32),
    compiler_params=pltpu.CompilerParams(
        dimension_semantics=("arbitrary",),
    ),
)


def _gather_body(tab_hbm, idx_hbm, out_hbm, idx_v, rows_v, sem):
    wid = lax.axis_index("s") * 2 + lax.axis_index("c")
    base = wid * _CHUNK
    pltpu.sync_copy(idx_hbm.at[pl.ds(base, _CHUNK)], idx_v)
    pltpu.async_copy(tab_hbm.at[idx_v], rows_v, sem).wait()
    pltpu.sync_copy(rows_v, out_hbm.at[pl.ds(base, _CHUNK)])


@functools.cache
def _gather_call():
    return pl.kernel(
        _gather_body,
        mesh=plsc.VectorSubcoreMesh(core_axis_name="c", subcore_axis_name="s"),
        out_type=jax.ShapeDtypeStruct((B * N_PAD,), jnp.float32),
        scratch_types=[
            pltpu.VMEM((_CHUNK,), jnp.int32),
            pltpu.VMEM((_CHUNK,), jnp.float32),
            pltpu.SemaphoreType.DMA,
        ],
    )


def kernel(h_e, h_c, h_share, pos_emb, W_hid, b_hid, ln_g, ln_b, W_rel, b_rel):
    band = _band_call(
        jnp.asarray(G_MAT, dtype=jnp.float32),
        h_e + h_share, h_c + h_share, pos_emb, W_hid,
        b_hid.reshape(1, H), ln_g.reshape(1, H), ln_b.reshape(1, H),
        W_rel, b_rel.reshape(1, 1),
    )
    flat = _gather_call()(band.reshape(B * BAND_STRIDE), jnp.asarray(IDX_FLAT))
    out = flat.reshape(B, N_PAD)[:, :N_PAIR]
    return out, jnp.asarray(POS_PAIRS)
